# baseline (speedup 1.0000x reference)
import functools

import jax
import jax.numpy as jnp
from jax.experimental import pallas as pl
from jax.experimental.pallas import tpu as pltpu


def _body(x_ref, xb_ref, w1_ref, s1_ref, b1_ref,
          w2_ref, s2_ref, b2_ref, w3_ref, s3_ref, b3_ref,
          o_ref,
          top_ref, w1_s, s1_s, b1_s, w2_s, s2_s, b2_s, w3_s, s3_s, b3_s,
          sems, *, HB, G, W, N, Cin, P):
    g = pl.program_id(0)
    rows = HB * W * N
    hrow = W * N

    @pl.when(g == 0)
    def _load_params():
        pairs = ((w1_ref, w1_s), (s1_ref, s1_s), (b1_ref, b1_s),
                 (w2_ref, w2_s), (s2_ref, s2_s), (b2_ref, b2_s),
                 (w3_ref, w3_s), (s3_ref, s3_s), (b3_ref, b3_s))
        copies = [pltpu.make_async_copy(src, dst, sems.at[i])
                  for i, (src, dst) in enumerate(pairs)]
        for c in copies:
            c.start()
        for c in copies:
            c.wait()

    xm = x_ref[...].reshape(rows, Cin)
    w1 = w1_s[...]

    def conv1(v):
        h = jnp.dot(v, w1, preferred_element_type=jnp.float32)
        return jnp.maximum(h * s1_s[...] + b1_s[...], 0.0)

    h1_mid = conv1(xm).astype(jnp.bfloat16)
    h1_bot = jnp.where(g == G - 1, 0.0,
                       conv1(xb_ref[...].reshape(hrow, Cin))
                       ).astype(jnp.bfloat16)
    h1_top = jnp.where(g == 0, 0.0, top_ref[...])
    top_ref[...] = h1_mid[(HB - 1) * hrow:]

    h1 = jnp.concatenate([h1_top, h1_mid, h1_bot], axis=0)
    h1m = h1.reshape(HB + 2, W, N, P)
    zcol = jnp.zeros((HB + 2, 1, N, P), h1m.dtype)
    left = jnp.concatenate([zcol, h1m[:, :W - 1]], axis=1)
    right = jnp.concatenate([h1m[:, 1:], zcol], axis=1)
    hcat = jnp.concatenate([left, h1m, right], axis=3)

    w2 = w2_s[...].astype(jnp.bfloat16)
    acc = jnp.dot(hcat[:HB].reshape(rows, 3 * P), w2[0],
                  preferred_element_type=jnp.float32)
    acc = acc + jnp.dot(hcat[1:HB + 1].reshape(rows, 3 * P), w2[1],
                        preferred_element_type=jnp.float32)
    acc = acc + jnp.dot(hcat[2:].reshape(rows, 3 * P), w2[2],
                        preferred_element_type=jnp.float32)
    h2 = jnp.maximum(acc * s2_s[...] + b2_s[...], 0.0)

    h3 = jnp.dot(h2.astype(jnp.bfloat16), w3_s[...].astype(jnp.bfloat16),
                 preferred_element_type=jnp.float32)
    h3 = h3 * s3_s[...] + b3_s[...]
    out = jnp.maximum(h3 + xm, 0.0)
    o_ref[...] = out.reshape(HB, hrow, Cin).astype(o_ref.dtype)


def kernel(x_nhwc, w1, s1, b1, w2, s2, b2, w3, s3, b3):
    N, H, W, Cin = x_nhwc.shape
    P = w1.shape[1]
    HB = 7
    G = H // HB

    x3 = jnp.transpose(x_nhwc, (1, 2, 0, 3)).reshape(H, W * N, Cin)
    w2c = w2.reshape(3, 3 * P, P)

    anyspec = pl.BlockSpec(memory_space=pl.ANY)
    body = functools.partial(_body, HB=HB, G=G, W=W, N=N, Cin=Cin, P=P)

    out3 = pl.pallas_call(
        body,
        out_shape=jax.ShapeDtypeStruct((H, W * N, Cin), x_nhwc.dtype),
        grid=(G,),
        in_specs=[
            pl.BlockSpec((HB, W * N, Cin), lambda g: (g, 0, 0)),
            pl.BlockSpec((1, W * N, Cin),
                         lambda g: (jnp.minimum(g * HB + HB, H - 1), 0, 0)),
        ] + [anyspec] * 9,
        out_specs=pl.BlockSpec((HB, W * N, Cin), lambda g: (g, 0, 0)),
        scratch_shapes=[
            pltpu.VMEM((W * N, P), jnp.bfloat16),
            pltpu.VMEM((Cin, P), jnp.float32),
            pltpu.VMEM((1, P), jnp.float32),
            pltpu.VMEM((1, P), jnp.float32),
            pltpu.VMEM((3, 3 * P, P), jnp.float32),
            pltpu.VMEM((1, P), jnp.float32),
            pltpu.VMEM((1, P), jnp.float32),
            pltpu.VMEM((P, Cin), jnp.float32),
            pltpu.VMEM((1, Cin), jnp.float32),
            pltpu.VMEM((1, Cin), jnp.float32),
            pltpu.SemaphoreType.DMA((9,)),
        ],
        compiler_params=pltpu.CompilerParams(
            dimension_semantics=("arbitrary",),
            vmem_limit_bytes=55 * 1024 * 1024),
    )(x3, x3,
      w1, s1, b1,
      w2c, s2, b2,
      w3, s3, b3)

    return jnp.transpose(out3.reshape(H, W, N, Cin), (2, 0, 1, 3))


# speedup vs baseline: 1.0757x; 1.0757x over previous
import functools

import jax
import jax.numpy as jnp
from jax.experimental import pallas as pl
from jax.experimental.pallas import tpu as pltpu


def _body(x_ref, xb_ref, w1_ref, s1_ref, b1_ref,
          w2_ref, s2_ref, b2_ref, w3_ref, s3_ref, b3_ref,
          o_ref, top_ref, *, HB, G, W, N, Cin, P):
    g = pl.program_id(0)
    rows = HB * W * N
    hrow = W * N

    xm = x_ref[...].reshape(rows, Cin)
    w1 = w1_ref[...]

    def conv1(v):
        h = jnp.dot(v, w1, preferred_element_type=jnp.float32)
        return jnp.maximum(h * s1_ref[...] + b1_ref[...], 0.0)

    h1_mid = conv1(xm).astype(jnp.bfloat16)
    h1_bot = jnp.where(g == G - 1, 0.0,
                       conv1(xb_ref[...].reshape(hrow, Cin))
                       ).astype(jnp.bfloat16)
    h1_top = jnp.where(g == 0, 0.0, top_ref[...])
    top_ref[...] = h1_mid[(HB - 1) * hrow:]

    h1 = jnp.concatenate([h1_top, h1_mid, h1_bot], axis=0)
    h1m = h1.reshape(HB + 2, W, N, P)
    zcol = jnp.zeros((HB + 2, 1, N, P), h1m.dtype)
    left = jnp.concatenate([zcol, h1m[:, :W - 1]], axis=1)
    right = jnp.concatenate([h1m[:, 1:], zcol], axis=1)
    hcat = jnp.concatenate([left, h1m, right], axis=3)

    w2 = w2_ref[...].astype(jnp.bfloat16)
    acc = jnp.dot(hcat[:HB].reshape(rows, 3 * P), w2[0],
                  preferred_element_type=jnp.float32)
    acc = acc + jnp.dot(hcat[1:HB + 1].reshape(rows, 3 * P), w2[1],
                        preferred_element_type=jnp.float32)
    acc = acc + jnp.dot(hcat[2:].reshape(rows, 3 * P), w2[2],
                        preferred_element_type=jnp.float32)
    h2 = jnp.maximum(acc * s2_ref[...] + b2_ref[...], 0.0)

    h3 = jnp.dot(h2.astype(jnp.bfloat16), w3_ref[...].astype(jnp.bfloat16),
                 preferred_element_type=jnp.float32)
    h3 = h3 * s3_ref[...] + b3_ref[...]
    out = jnp.maximum(h3 + xm, 0.0)
    o_ref[...] = out.reshape(HB, hrow, Cin).astype(o_ref.dtype)


def kernel(x_nhwc, w1, s1, b1, w2, s2, b2, w3, s3, b3):
    N, H, W, Cin = x_nhwc.shape
    P = w1.shape[1]
    HB = 7
    G = H // HB

    x3 = jnp.transpose(x_nhwc, (1, 2, 0, 3)).reshape(H, W * N, Cin)
    w2c = w2.reshape(3, 3 * P, P)

    full = lambda a: pl.BlockSpec(a.shape, lambda g: (0,) * a.ndim)
    body = functools.partial(_body, HB=HB, G=G, W=W, N=N, Cin=Cin, P=P)

    out3 = pl.pallas_call(
        body,
        out_shape=jax.ShapeDtypeStruct((H, W * N, Cin), x_nhwc.dtype),
        grid=(G,),
        in_specs=[
            pl.BlockSpec((HB, W * N, Cin), lambda g: (g, 0, 0)),
            pl.BlockSpec((1, W * N, Cin),
                         lambda g: (jnp.minimum(g * HB + HB, H - 1), 0, 0)),
            full(w1), full(s1), full(b1),
            full(w2c), full(s2), full(b2),
            full(w3), full(s3), full(b3),
        ],
        out_specs=pl.BlockSpec((HB, W * N, Cin), lambda g: (g, 0, 0)),
        scratch_shapes=[pltpu.VMEM((W * N, P), jnp.bfloat16)],
        compiler_params=pltpu.CompilerParams(
            dimension_semantics=("arbitrary",),
            vmem_limit_bytes=55 * 1024 * 1024),
    )(x3, x3,
      w1, s1, b1,
      w2c, s2, b2,
      w3, s3, b3)

    return jnp.transpose(out3.reshape(H, W, N, Cin), (2, 0, 1, 3))
